# baseline (speedup 1.0000x reference)
import functools

import jax
import jax.numpy as jnp
from jax import lax
from jax.experimental import pallas as pl
from jax.experimental.pallas import tpu as pltpu
from jax.experimental.pallas import tpu_sc as plsc

N = 10000
E = 320000
D = 128

NC = 2
NS = 16
GROUP = 128
GPW = 80
E_PAD = NC * NS * GPW * GROUP
N_ACC = 10240
RPT = N_ACC // NS

GPW0 = 80
GPW1 = 80


QB = 16


def _sc_seg_sum_body(want_deg, gpw0, gpw1, tab_hbm, src_hbm, dst_hbm, z2_hbm,
                     z1_hbm, o1_hbm, p0_hbm, p1_hbm, g0_hbm, g1_hbm,
                     acc_s, deg_s, sidx_v, didx_v, rows0_v, rows1_v, zero1_v,
                     ones_v, sem0, sem1):
    cid = lax.axis_index("c")
    sid = lax.axis_index("s")
    base = sid * RPT

    def init_tile():
        pltpu.sync_copy(z2_hbm, rows0_v)
        for k in range(RPT // 128):
            pltpu.sync_copy(rows0_v, acc_s.at[pl.ds(base + k * 128, 128)])
        if want_deg:
            pltpu.sync_copy(o1_hbm, ones_v)
            pltpu.sync_copy(z1_hbm, zero1_v)
            pltpu.sync_copy(zero1_v, deg_s.at[pl.ds(base, RPT)])

    if gpw1 == 0:
        @pl.when(cid == 0)
        def _():
            init_tile()
    else:
        init_tile()
    plsc.subcore_barrier()

    def gather(idx_row, rows_v, sem):
        pltpu.async_copy(tab_hbm.at[sidx_v.at[idx_row]], rows_v, sem)

    def drain(idx_row, rows_v, sem):
        pltpu.make_async_copy(tab_hbm.at[sidx_v.at[idx_row]], rows_v,
                              sem).wait()

    def scatter(idx_row, rows_v):
        pltpu.sync_copy(rows_v, acc_s.at[didx_v.at[idx_row]], add=True)
        if want_deg:
            pltpu.sync_copy(ones_v, deg_s.at[didx_v.at[idx_row]], add=True)

    def do_edges(gbase, nbatch):
        assert nbatch * QB in (gpw0, gpw1)
        for h in range(nbatch):
            pltpu.sync_copy(src_hbm.at[pl.ds(gbase + h * QB, QB)], sidx_v)
            pltpu.sync_copy(dst_hbm.at[pl.ds(gbase + h * QB, QB)], didx_v)
            gather(0, rows0_v, sem0)

            def pair(i, carry):
                g0 = 2 * i
                gather(g0 + 1, rows1_v, sem1)
                drain(g0, rows0_v, sem0)
                scatter(g0, rows0_v)
                gather(g0 + 2, rows0_v, sem0)
                drain(g0 + 1, rows1_v, sem1)
                scatter(g0 + 1, rows1_v)
                return carry

            lax.fori_loop(0, QB // 2 - 1, pair, 0)
            gather(QB - 1, rows1_v, sem1)
            drain(QB - 2, rows0_v, sem0)
            scatter(QB - 2, rows0_v)
            drain(QB - 1, rows1_v, sem1)
            scatter(QB - 1, rows1_v)

    if gpw0:
        @pl.when(cid == 0)
        def _():
            do_edges(sid * gpw0, gpw0 // QB)

    if gpw1:
        @pl.when(cid == 1)
        def _():
            do_edges(NS * gpw0 + sid * gpw1, gpw1 // QB)

    plsc.subcore_barrier()

    @pl.when(cid == 0)
    def _():
        pltpu.sync_copy(acc_s.at[pl.ds(base, RPT)], p0_hbm.at[pl.ds(base, RPT)])
        if want_deg:
            pltpu.sync_copy(deg_s.at[pl.ds(base, RPT)],
                            g0_hbm.at[pl.ds(base, RPT)])

    if gpw1:
        @pl.when(cid == 1)
        def _():
            pltpu.sync_copy(acc_s.at[pl.ds(base, RPT)],
                            p1_hbm.at[pl.ds(base, RPT)])
            if want_deg:
                pltpu.sync_copy(deg_s.at[pl.ds(base, RPT)],
                                g1_hbm.at[pl.ds(base, RPT)])


def _sc_out_type(want_deg, gpw1):
    mat = jax.ShapeDtypeStruct((N_ACC, D), jnp.float32)
    vec = jax.ShapeDtypeStruct((N_ACC,), jnp.float32)
    out = [mat]
    if gpw1:
        out.append(mat)
    if want_deg:
        out.append(vec)
        if gpw1:
            out.append(vec)
    return out


def _sc_body_with_outputs(want_deg, gpw0, gpw1):
    n_out = len(_sc_out_type(want_deg, gpw1))

    def body(*refs):
        ins = refs[:6]
        outs = list(refs[6:6 + n_out])
        scratch = refs[6 + n_out:]
        p0 = outs.pop(0)
        p1 = outs.pop(0) if gpw1 else None
        g0 = outs.pop(0) if want_deg else None
        g1 = outs.pop(0) if (want_deg and gpw1) else None
        return _sc_seg_sum_body(want_deg, gpw0, gpw1, *ins, p0, p1, g0, g1,
                                *scratch)

    return body


@functools.cache
def _sc_seg_sum(want_deg, gpw0=GPW, gpw1=GPW):
    mesh = plsc.VectorSubcoreMesh(core_axis_name="c", subcore_axis_name="s",
                                  num_cores=NC, num_subcores=NS)
    return pl.kernel(
        _sc_body_with_outputs(want_deg, gpw0, gpw1),
        out_type=_sc_out_type(want_deg, gpw1),
        mesh=mesh,
        scratch_types=[
            pltpu.VMEM_SHARED((N_ACC, D), jnp.float32),
            pltpu.VMEM_SHARED((N_ACC,), jnp.float32),
            pltpu.VMEM((QB, GROUP), jnp.int32),
            pltpu.VMEM((QB, GROUP), jnp.int32),
            pltpu.VMEM((GROUP, D), jnp.float32),
            pltpu.VMEM((GROUP, D), jnp.float32),
            pltpu.VMEM((RPT,), jnp.float32),
            pltpu.VMEM((GROUP,), jnp.float32),
            pltpu.SemaphoreType.DMA,
            pltpu.SemaphoreType.DMA,
        ],
    )


_BLK = 400


def _tc_pre_body(h_ref, ws_ref, b_ref, o_ref):
    o_ref[...] = jnp.dot(h_ref[...], ws_ref[...],
                         preferred_element_type=jnp.float32) + b_ref[...]


def _tc_combine_body(relu, two, *refs):
    if two:
        pre_ref, p0_ref, p1_ref, d0_ref, d1_ref, wn_ref, o_ref = refs
        psum = p0_ref[...] + p1_ref[...]
        dsum = d0_ref[...] + d1_ref[...]
    else:
        pre_ref, p0_ref, d0_ref, wn_ref, o_ref = refs
        psum = p0_ref[...]
        dsum = d0_ref[...]
    agg = psum * (1.0 / jnp.maximum(dsum, 1.0))
    o = pre_ref[...] + jnp.dot(agg, wn_ref[...],
                               preferred_element_type=jnp.float32)
    if relu:
        o = jnp.maximum(o, 0.0)
    o_ref[...] = o


def _tc_specs():
    mat = pl.BlockSpec((_BLK, D), lambda i: (i, 0))
    vec = pl.BlockSpec((_BLK, 1), lambda i: (i, 0))
    wspec = pl.BlockSpec((D, D), lambda i: (0, 0))
    bspec = pl.BlockSpec((1, D), lambda i: (0, 0))
    return mat, vec, wspec, bspec


@functools.cache
def _tc_pre():
    mat, vec, wspec, bspec = _tc_specs()
    return pl.pallas_call(
        _tc_pre_body,
        grid=(N // _BLK,),
        in_specs=[mat, wspec, bspec],
        out_specs=mat,
        out_shape=jax.ShapeDtypeStruct((N, D), jnp.float32),
    )


@functools.cache
def _tc_combine(relu, two):
    mat, vec, wspec, bspec = _tc_specs()
    if two:
        in_specs = [mat, mat, mat, vec, vec, wspec]
    else:
        in_specs = [mat, mat, vec, wspec]
    return pl.pallas_call(
        functools.partial(_tc_combine_body, relu, two),
        grid=(N // _BLK,),
        in_specs=in_specs,
        out_specs=mat,
        out_shape=jax.ShapeDtypeStruct((N, D), jnp.float32),
    )


def kernel(x, edge_index, W_self1, W_neigh1, b1, W_self2, W_neigh2, b2):
    src = edge_index[0]
    dst = edge_index[1]
    npad = E_PAD - E
    ar = jnp.arange(npad, dtype=jnp.int32)
    pad_src = (ar * 131) % N
    pad_dst = N + ar % (N_ACC - N)
    src2 = jnp.concatenate([src, pad_src]).reshape(E_PAD // GROUP, GROUP)
    dst2 = jnp.concatenate([dst, pad_dst]).reshape(E_PAD // GROUP, GROUP)
    z2 = jnp.zeros((128, D), jnp.float32)
    z1 = jnp.zeros((RPT,), jnp.float32)
    o1 = jnp.ones((GROUP,), jnp.float32)

    two = GPW1 > 0
    pre1 = _tc_pre()(x, W_self1, b1.reshape(1, D))
    if two:
        p0, p1, g0, g1 = _sc_seg_sum(True, GPW0, GPW1)(x, src2, dst2,
                                                       z2, z1, o1)
        h = _tc_combine(True, True)(pre1, p0, p1, g0[:, None], g1[:, None],
                                    W_neigh1)
    else:
        p0, g0 = _sc_seg_sum(True, GPW0, GPW1)(x, src2, dst2, z2, z1, o1)
        h = _tc_combine(True, False)(pre1, p0, g0[:, None], W_neigh1)

    pre2 = _tc_pre()(h, W_self2, b2.reshape(1, D))
    if two:
        q0, q1 = _sc_seg_sum(False, GPW0, GPW1)(h, src2, dst2, z2, z1, o1)
        out = _tc_combine(False, True)(pre2, q0, q1, g0[:, None], g1[:, None],
                                       W_neigh2)
    else:
        q0, = _sc_seg_sum(False, GPW0, GPW1)(h, src2, dst2, z2, z1, o1)
        out = _tc_combine(False, False)(pre2, q0, g0[:, None], W_neigh2)
    return out


# speedup vs baseline: 1.0103x; 1.0103x over previous
import functools

import jax
import jax.numpy as jnp
import numpy as np
from jax import lax
from jax.experimental import pallas as pl
from jax.experimental.pallas import tpu as pltpu
from jax.experimental.pallas import tpu_sc as plsc

N = 10000
E = 320000
D = 128

NC = 2
NS = 16
GROUP = 128
GPW = 80
E_PAD = NC * NS * GPW * GROUP
N_ACC = 10240
RPT = N_ACC // NS

GPW0 = 80
GPW1 = 80

_NPAD = E_PAD - E
_PAD_SRC = np.asarray((np.arange(_NPAD) * 131) % N, dtype=np.int32)
_PAD_DST = np.asarray(N + np.arange(_NPAD) % (N_ACC - N), dtype=np.int32)


QB = 16


def _sc_seg_sum_body(want_deg, gpw0, gpw1, tab_hbm, src_hbm, dst_hbm, z2_hbm,
                     z1_hbm, o1_hbm, p0_hbm, p1_hbm, g0_hbm, g1_hbm,
                     acc_s, deg_s, sidx_v, didx_v, rows0_v, rows1_v, zero1_v,
                     ones_v, sem0, sem1):
    cid = lax.axis_index("c")
    sid = lax.axis_index("s")
    base = sid * RPT

    def init_tile():
        pltpu.sync_copy(z2_hbm, rows0_v)
        for k in range(RPT // 128):
            pltpu.sync_copy(rows0_v, acc_s.at[pl.ds(base + k * 128, 128)])
        if want_deg:
            pltpu.sync_copy(o1_hbm, ones_v)
            pltpu.sync_copy(z1_hbm, zero1_v)
            pltpu.sync_copy(zero1_v, deg_s.at[pl.ds(base, RPT)])

    if gpw1 == 0:
        @pl.when(cid == 0)
        def _():
            init_tile()
    else:
        init_tile()
    plsc.subcore_barrier()

    def gather(idx_row, rows_v, sem):
        pltpu.async_copy(tab_hbm.at[sidx_v.at[idx_row]], rows_v, sem)

    def drain(idx_row, rows_v, sem):
        pltpu.make_async_copy(tab_hbm.at[sidx_v.at[idx_row]], rows_v,
                              sem).wait()

    def scatter(idx_row, rows_v):
        pltpu.sync_copy(rows_v, acc_s.at[didx_v.at[idx_row]], add=True)
        if want_deg:
            pltpu.sync_copy(ones_v, deg_s.at[didx_v.at[idx_row]], add=True)

    def do_edges(gbase, nbatch):
        assert nbatch * QB in (gpw0, gpw1)
        for h in range(nbatch):
            pltpu.sync_copy(src_hbm.at[pl.ds(gbase + h * QB, QB)], sidx_v)
            pltpu.sync_copy(dst_hbm.at[pl.ds(gbase + h * QB, QB)], didx_v)
            gather(0, rows0_v, sem0)

            def pair(i, carry):
                g0 = 2 * i
                gather(g0 + 1, rows1_v, sem1)
                drain(g0, rows0_v, sem0)
                scatter(g0, rows0_v)
                gather(g0 + 2, rows0_v, sem0)
                drain(g0 + 1, rows1_v, sem1)
                scatter(g0 + 1, rows1_v)
                return carry

            lax.fori_loop(0, QB // 2 - 1, pair, 0)
            gather(QB - 1, rows1_v, sem1)
            drain(QB - 2, rows0_v, sem0)
            scatter(QB - 2, rows0_v)
            drain(QB - 1, rows1_v, sem1)
            scatter(QB - 1, rows1_v)

    if gpw0:
        @pl.when(cid == 0)
        def _():
            do_edges(sid * gpw0, gpw0 // QB)

    if gpw1:
        @pl.when(cid == 1)
        def _():
            do_edges(NS * gpw0 + sid * gpw1, gpw1 // QB)

    plsc.subcore_barrier()

    @pl.when(cid == 0)
    def _():
        pltpu.sync_copy(acc_s.at[pl.ds(base, RPT)], p0_hbm.at[pl.ds(base, RPT)])
        if want_deg:
            pltpu.sync_copy(deg_s.at[pl.ds(base, RPT)],
                            g0_hbm.at[pl.ds(base, RPT)])

    if gpw1:
        @pl.when(cid == 1)
        def _():
            pltpu.sync_copy(acc_s.at[pl.ds(base, RPT)],
                            p1_hbm.at[pl.ds(base, RPT)])
            if want_deg:
                pltpu.sync_copy(deg_s.at[pl.ds(base, RPT)],
                                g1_hbm.at[pl.ds(base, RPT)])


def _sc_out_type(want_deg, gpw1):
    mat = jax.ShapeDtypeStruct((N_ACC, D), jnp.float32)
    vec = jax.ShapeDtypeStruct((N_ACC,), jnp.float32)
    out = [mat]
    if gpw1:
        out.append(mat)
    if want_deg:
        out.append(vec)
        if gpw1:
            out.append(vec)
    return out


def _sc_body_with_outputs(want_deg, gpw0, gpw1):
    n_out = len(_sc_out_type(want_deg, gpw1))

    def body(*refs):
        ins = refs[:6]
        outs = list(refs[6:6 + n_out])
        scratch = refs[6 + n_out:]
        p0 = outs.pop(0)
        p1 = outs.pop(0) if gpw1 else None
        g0 = outs.pop(0) if want_deg else None
        g1 = outs.pop(0) if (want_deg and gpw1) else None
        return _sc_seg_sum_body(want_deg, gpw0, gpw1, *ins, p0, p1, g0, g1,
                                *scratch)

    return body


@functools.cache
def _sc_seg_sum(want_deg, gpw0=GPW, gpw1=GPW):
    mesh = plsc.VectorSubcoreMesh(core_axis_name="c", subcore_axis_name="s",
                                  num_cores=NC, num_subcores=NS)
    return pl.kernel(
        _sc_body_with_outputs(want_deg, gpw0, gpw1),
        out_type=_sc_out_type(want_deg, gpw1),
        mesh=mesh,
        scratch_types=[
            pltpu.VMEM_SHARED((N_ACC, D), jnp.float32),
            pltpu.VMEM_SHARED((N_ACC,), jnp.float32),
            pltpu.VMEM((QB, GROUP), jnp.int32),
            pltpu.VMEM((QB, GROUP), jnp.int32),
            pltpu.VMEM((GROUP, D), jnp.float32),
            pltpu.VMEM((GROUP, D), jnp.float32),
            pltpu.VMEM((RPT,), jnp.float32),
            pltpu.VMEM((GROUP,), jnp.float32),
            pltpu.SemaphoreType.DMA,
            pltpu.SemaphoreType.DMA,
        ],
    )


_BLK = 400


def _tc_pre_body(h_ref, ws_ref, b_ref, o_ref):
    o_ref[...] = jnp.dot(h_ref[...], ws_ref[...],
                         preferred_element_type=jnp.float32) + b_ref[...]


def _tc_combine_body(relu, two, *refs):
    if two:
        pre_ref, p0_ref, p1_ref, r_ref, wn_ref, o_ref = refs
        psum = p0_ref[...] + p1_ref[...]
    else:
        pre_ref, p0_ref, r_ref, wn_ref, o_ref = refs
        psum = p0_ref[...]
    rcol = jnp.transpose(r_ref[...].reshape(1, _BLK))
    agg = psum * rcol
    o = pre_ref[...] + jnp.dot(agg, wn_ref[...],
                               preferred_element_type=jnp.float32)
    if relu:
        o = jnp.maximum(o, 0.0)
    o_ref[...] = o


@functools.cache
def _tc_pre():
    mat = pl.BlockSpec((400, D), lambda i: (i, 0))
    return pl.pallas_call(
        _tc_pre_body,
        grid=(N // 400,),
        in_specs=[mat,
                  pl.BlockSpec((D, D), lambda i: (0, 0)),
                  pl.BlockSpec((1, D), lambda i: (0, 0))],
        out_specs=mat,
        out_shape=jax.ShapeDtypeStruct((N, D), jnp.float32),
    )


@functools.cache
def _tc_combine(relu, two):
    mat = pl.BlockSpec((_BLK, D), lambda i: (i, 0))
    drow = pl.BlockSpec((1, 1, _BLK), lambda i: (i, 0, 0))
    wspec = pl.BlockSpec((D, D), lambda i: (0, 0))
    if two:
        in_specs = [mat, mat, mat, drow, wspec]
    else:
        in_specs = [mat, mat, drow, wspec]
    return pl.pallas_call(
        functools.partial(_tc_combine_body, relu, two),
        grid=(N // _BLK,),
        in_specs=in_specs,
        out_specs=mat,
        out_shape=jax.ShapeDtypeStruct((N, D), jnp.float32),
    )


def kernel(x, edge_index, W_self1, W_neigh1, b1, W_self2, W_neigh2, b2):
    src = edge_index[0]
    dst = edge_index[1]
    src2 = jnp.concatenate([src, _PAD_SRC]).reshape(E_PAD // GROUP, GROUP)
    dst2 = jnp.concatenate([dst, _PAD_DST]).reshape(E_PAD // GROUP, GROUP)
    z2 = jnp.zeros((128, D), jnp.float32)
    z1 = jnp.zeros((RPT,), jnp.float32)
    o1 = jnp.ones((GROUP,), jnp.float32)

    two = GPW1 > 0
    pre1 = _tc_pre()(x, W_self1, b1.reshape(1, D))
    if two:
        p0, p1, g0, g1 = _sc_seg_sum(True, GPW0, GPW1)(x, src2, dst2,
                                                       z2, z1, o1)
        deg = g0[:N] + g1[:N]
    else:
        p0, g0 = _sc_seg_sum(True, GPW0, GPW1)(x, src2, dst2, z2, z1, o1)
        deg = g0[:N]
    rd = (1.0 / jnp.maximum(deg, 1.0)).reshape(N // _BLK, 1, _BLK)
    if two:
        h = _tc_combine(True, True)(pre1, p0, p1, rd, W_neigh1)
    else:
        h = _tc_combine(True, False)(pre1, p0, rd, W_neigh1)

    pre2 = _tc_pre()(h, W_self2, b2.reshape(1, D))
    if two:
        q0, q1 = _sc_seg_sum(False, GPW0, GPW1)(h, src2, dst2, z2, z1, o1)
        out = _tc_combine(False, True)(pre2, q0, q1, rd, W_neigh2)
    else:
        q0, = _sc_seg_sum(False, GPW0, GPW1)(h, src2, dst2, z2, z1, o1)
        out = _tc_combine(False, False)(pre2, q0, rd, W_neigh2)
    return out


# speedup vs baseline: 1.0285x; 1.0180x over previous
import functools

import jax
import jax.numpy as jnp
import numpy as np
from jax import lax
from jax.experimental import pallas as pl
from jax.experimental.pallas import tpu as pltpu
from jax.experimental.pallas import tpu_sc as plsc

N = 10000
E = 320000
D = 128

NC = 2
NS = 16
NW = NC * NS
GROUP = 128
G_REAL = E // GROUP
GPW = 80
G_TOT = NW * GPW
N_ACC = 10240
RPT = N_ACC // NS
QB = 16

_NPAD = (G_TOT - G_REAL) * GROUP
_PAD_SRC = np.asarray((np.arange(_NPAD) * 131) % N,
                      dtype=np.int32).reshape(-1, GROUP)
_PAD_DST = np.asarray(N + np.arange(_NPAD) % (N_ACC - N),
                      dtype=np.int32).reshape(-1, GROUP)


def _sc_body_fixed(want_deg):
    def body(tab_hbm, eidx_hbm, ps_hbm, pd_hbm, z2_hbm, z1_hbm, o1_hbm,
             *rest):
        n_out = 4 if want_deg else 2
        outs = list(rest[:n_out])
        scratch = rest[n_out:]
        p0 = outs.pop(0)
        p1 = outs.pop(0)
        g0 = outs.pop(0) if want_deg else None
        g1 = outs.pop(0) if want_deg else None
        return _sc_seg_sum_body(want_deg, tab_hbm, eidx_hbm, ps_hbm,
                                pd_hbm, z2_hbm, z1_hbm, o1_hbm,
                                p0, p1, g0, g1, *scratch)
    return body


def _sc_seg_sum_body(want_deg, tab_hbm, eidx_hbm, ps_hbm, pd_hbm,
                          z2_hbm, z1_hbm, o1_hbm, p0_hbm, p1_hbm,
                          g0_hbm, g1_hbm, acc_s, deg_s, sidx_v, didx_v,
                          rows0_v, rows1_v, zero1_v, ones_v, sem0, sem1):
    cid = lax.axis_index("c")
    sid = lax.axis_index("s")
    w = cid * NS + sid
    base = sid * RPT

    pltpu.sync_copy(z2_hbm, rows0_v)
    for k in range(RPT // 128):
        pltpu.sync_copy(rows0_v, acc_s.at[pl.ds(base + k * 128, 128)])
    if want_deg:
        pltpu.sync_copy(o1_hbm, ones_v)
        pltpu.sync_copy(z1_hbm, zero1_v)
        pltpu.sync_copy(zero1_v, deg_s.at[pl.ds(base, RPT)])
    plsc.subcore_barrier()

    def gather(idx_row, rows_v, sem):
        pltpu.async_copy(tab_hbm.at[sidx_v.at[idx_row]], rows_v, sem)

    def drain(idx_row, rows_v, sem):
        pltpu.make_async_copy(tab_hbm.at[sidx_v.at[idx_row]], rows_v,
                              sem).wait()

    def scatter(idx_row, rows_v):
        pltpu.sync_copy(rows_v, acc_s.at[didx_v.at[idx_row]], add=True)
        if want_deg:
            pltpu.sync_copy(ones_v, deg_s.at[didx_v.at[idx_row]], add=True)

    def run_batch(src_sl, dst_sl, nrows):
        pltpu.sync_copy(src_sl, sidx_v.at[pl.ds(0, nrows)])
        pltpu.sync_copy(dst_sl, didx_v.at[pl.ds(0, nrows)])
        gather(0, rows0_v, sem0)

        def pair(i, carry):
            g0 = 2 * i
            gather(g0 + 1, rows1_v, sem1)
            drain(g0, rows0_v, sem0)
            scatter(g0, rows0_v)
            gather(g0 + 2, rows0_v, sem0)
            drain(g0 + 1, rows1_v, sem1)
            scatter(g0 + 1, rows1_v)
            return carry

        if nrows > 2:
            lax.fori_loop(0, nrows // 2 - 1, pair, 0)
        gather(nrows - 1, rows1_v, sem1)
        drain(nrows - 2, rows0_v, sem0)
        scatter(nrows - 2, rows0_v)
        drain(nrows - 1, rows1_v, sem1)
        scatter(nrows - 1, rows1_v)

    @pl.when(w != NW - 1)
    def _():
        rb = w * GPW
        for h in range(GPW // QB):
            s = rb + h * QB
            run_batch(eidx_hbm.at[0, pl.ds(s, QB)],
                      eidx_hbm.at[1, pl.ds(s, QB)], QB)

    @pl.when(w == NW - 1)
    def _():
        run_batch(eidx_hbm.at[0, pl.ds(G_REAL - 20, 16)],
                  eidx_hbm.at[1, pl.ds(G_REAL - 20, 16)], 16)
        run_batch(eidx_hbm.at[0, pl.ds(G_REAL - 4, 4)],
                  eidx_hbm.at[1, pl.ds(G_REAL - 4, 4)], 4)
        for s, l in ((0, 16), (16, 16), (32, 16), (48, 12)):
            run_batch(ps_hbm.at[pl.ds(s, l)], pd_hbm.at[pl.ds(s, l)], l)

    plsc.subcore_barrier()

    @pl.when(cid == 0)
    def _():
        pltpu.sync_copy(acc_s.at[pl.ds(base, RPT)], p0_hbm.at[pl.ds(base, RPT)])
        if want_deg:
            pltpu.sync_copy(deg_s.at[pl.ds(base, RPT)],
                            g0_hbm.at[pl.ds(base, RPT)])

    @pl.when(cid == 1)
    def _():
        pltpu.sync_copy(acc_s.at[pl.ds(base, RPT)], p1_hbm.at[pl.ds(base, RPT)])
        if want_deg:
            pltpu.sync_copy(deg_s.at[pl.ds(base, RPT)],
                            g1_hbm.at[pl.ds(base, RPT)])


def _sc_out_type(want_deg):
    mat = jax.ShapeDtypeStruct((N_ACC, D), jnp.float32)
    vec = jax.ShapeDtypeStruct((N_ACC,), jnp.float32)
    out = [mat, mat]
    if want_deg:
        out += [vec, vec]
    return out


@functools.cache
def _sc_seg_sum(want_deg):
    mesh = plsc.VectorSubcoreMesh(core_axis_name="c", subcore_axis_name="s",
                                  num_cores=NC, num_subcores=NS)
    return pl.kernel(
        _sc_body_fixed(want_deg),
        out_type=_sc_out_type(want_deg),
        mesh=mesh,
        scratch_types=[
            pltpu.VMEM_SHARED((N_ACC, D), jnp.float32),
            pltpu.VMEM_SHARED((N_ACC,), jnp.float32),
            pltpu.VMEM((QB, GROUP), jnp.int32),
            pltpu.VMEM((QB, GROUP), jnp.int32),
            pltpu.VMEM((GROUP, D), jnp.float32),
            pltpu.VMEM((GROUP, D), jnp.float32),
            pltpu.VMEM((RPT,), jnp.float32),
            pltpu.VMEM((GROUP,), jnp.float32),
            pltpu.SemaphoreType.DMA,
            pltpu.SemaphoreType.DMA,
        ],
    )


_BLK = 400


def _tc_pre_body(h_ref, ws_ref, b_ref, o_ref):
    o_ref[...] = jnp.dot(h_ref[...].astype(jnp.bfloat16),
                         ws_ref[...].astype(jnp.bfloat16),
                         preferred_element_type=jnp.float32) + b_ref[...]


def _tc_combine_body(relu, pre_ref, p0_ref, p1_ref, r_ref, wn_ref, o_ref):
    psum = p0_ref[...] + p1_ref[...]
    rcol = jnp.transpose(r_ref[...].reshape(1, _BLK))
    agg = psum * rcol
    o = pre_ref[...] + jnp.dot(agg.astype(jnp.bfloat16),
                               wn_ref[...].astype(jnp.bfloat16),
                               preferred_element_type=jnp.float32)
    if relu:
        o = jnp.maximum(o, 0.0)
    o_ref[...] = o


@functools.cache
def _tc_pre():
    mat = pl.BlockSpec((_BLK, D), lambda i: (i, 0))
    return pl.pallas_call(
        _tc_pre_body,
        grid=(N // _BLK,),
        in_specs=[mat,
                  pl.BlockSpec((D, D), lambda i: (0, 0)),
                  pl.BlockSpec((1, D), lambda i: (0, 0))],
        out_specs=mat,
        out_shape=jax.ShapeDtypeStruct((N, D), jnp.float32),
    )


@functools.cache
def _tc_combine(relu):
    mat = pl.BlockSpec((_BLK, D), lambda i: (i, 0))
    drow = pl.BlockSpec((1, 1, _BLK), lambda i: (i, 0, 0))
    wspec = pl.BlockSpec((D, D), lambda i: (0, 0))
    return pl.pallas_call(
        functools.partial(_tc_combine_body, relu),
        grid=(N // _BLK,),
        in_specs=[mat, mat, mat, drow, wspec],
        out_specs=mat,
        out_shape=jax.ShapeDtypeStruct((N, D), jnp.float32),
    )


def kernel(x, edge_index, W_self1, W_neigh1, b1, W_self2, W_neigh2, b2):
    eidx = edge_index.reshape(2, G_REAL, GROUP)
    ps = jnp.asarray(_PAD_SRC)
    pd = jnp.asarray(_PAD_DST)
    z2 = jnp.zeros((128, D), jnp.float32)
    z1 = jnp.zeros((RPT,), jnp.float32)
    o1 = jnp.ones((GROUP,), jnp.float32)

    pre1 = _tc_pre()(x, W_self1, b1.reshape(1, D))
    p0, p1, g0, g1 = _sc_seg_sum(True)(x, eidx, ps, pd, z2, z1, o1)
    rd = (1.0 / jnp.maximum(g0[:N] + g1[:N], 1.0)).reshape(N // _BLK, 1, _BLK)
    h = _tc_combine(True)(pre1, p0, p1, rd, W_neigh1)

    pre2 = _tc_pre()(h, W_self2, b2.reshape(1, D))
    q0, q1 = _sc_seg_sum(False)(h, eidx, ps, pd, z2, z1, o1)
    out = _tc_combine(False)(pre2, q0, q1, rd, W_neigh2)
    return out


# speedup vs baseline: 1.1103x; 1.0795x over previous
import functools

import jax
import jax.numpy as jnp
import numpy as np
from jax import lax
from jax.experimental import pallas as pl
from jax.experimental.pallas import tpu as pltpu
from jax.experimental.pallas import tpu_sc as plsc

N = 10000
E = 320000
D = 128

NC = 2
NS = 16
NW = NC * NS
GROUP = 128
G_REAL = E // GROUP
GPW = 80
G_TOT = NW * GPW
N_ACC = 10240
RPT = N_ACC // NS
QB = 16

_NPAD = (G_TOT - G_REAL) * GROUP
_PAD_SRC = np.asarray((np.arange(_NPAD) * 131) % N,
                      dtype=np.int32).reshape(-1, GROUP)
_PAD_DST = np.asarray(N + np.arange(_NPAD) % (N_ACC - N),
                      dtype=np.int32).reshape(-1, GROUP)


def _sc_body_fixed(want_deg):
    def body(tab_hbm, eidx_hbm, ps_hbm, pd_hbm, z2_hbm, z1_hbm, o1_hbm,
             *rest):
        n_out = 4 if want_deg else 2
        outs = list(rest[:n_out])
        scratch = rest[n_out:]
        p0 = outs.pop(0)
        p1 = outs.pop(0)
        g0 = outs.pop(0) if want_deg else None
        g1 = outs.pop(0) if want_deg else None
        return _sc_seg_sum_body(want_deg, tab_hbm, eidx_hbm, ps_hbm,
                                pd_hbm, z2_hbm, z1_hbm, o1_hbm,
                                p0, p1, g0, g1, *scratch)
    return body


def _sc_seg_sum_body(want_deg, tab_hbm, eidx_hbm, ps_hbm, pd_hbm,
                          z2_hbm, z1_hbm, o1_hbm, p0_hbm, p1_hbm,
                          g0_hbm, g1_hbm, acc_s, deg_s, sidx_v, didx_v,
                          rows0_v, rows1_v, zero1_v, ones_v, sem0, sem1):
    cid = lax.axis_index("c")
    sid = lax.axis_index("s")
    w = cid * NS + sid
    base = sid * RPT

    pltpu.sync_copy(z2_hbm, rows0_v)
    for k in range(RPT // 128):
        pltpu.sync_copy(rows0_v, acc_s.at[pl.ds(base + k * 128, 128)])
    if want_deg:
        pltpu.sync_copy(o1_hbm, ones_v)
        pltpu.sync_copy(z1_hbm, zero1_v)
        pltpu.sync_copy(zero1_v, deg_s.at[pl.ds(base, RPT)])
    plsc.subcore_barrier()

    def gather(idx_row, rows_v, sem):
        pltpu.async_copy(tab_hbm.at[sidx_v.at[idx_row]], rows_v, sem)

    def drain(idx_row, rows_v, sem):
        pltpu.make_async_copy(tab_hbm.at[sidx_v.at[idx_row]], rows_v,
                              sem).wait()

    def scatter(idx_row, rows_v):
        pltpu.sync_copy(rows_v, acc_s.at[didx_v.at[idx_row]], add=True)
        if want_deg:
            pltpu.sync_copy(ones_v, deg_s.at[didx_v.at[idx_row]], add=True)

    def run_batch(src_sl, dst_sl, nrows):
        pltpu.sync_copy(src_sl, sidx_v.at[pl.ds(0, nrows)])
        pltpu.sync_copy(dst_sl, didx_v.at[pl.ds(0, nrows)])
        gather(0, rows0_v, sem0)

        def pair(i, carry):
            g0 = 2 * i
            gather(g0 + 1, rows1_v, sem1)
            drain(g0, rows0_v, sem0)
            scatter(g0, rows0_v)
            gather(g0 + 2, rows0_v, sem0)
            drain(g0 + 1, rows1_v, sem1)
            scatter(g0 + 1, rows1_v)
            return carry

        if nrows > 2:
            lax.fori_loop(0, nrows // 2 - 1, pair, 0)
        gather(nrows - 1, rows1_v, sem1)
        drain(nrows - 2, rows0_v, sem0)
        scatter(nrows - 2, rows0_v)
        drain(nrows - 1, rows1_v, sem1)
        scatter(nrows - 1, rows1_v)

    @pl.when(w != NW - 1)
    def _():
        rb = w * GPW
        for h in range(GPW // QB):
            s = rb + h * QB
            run_batch(eidx_hbm.at[0, pl.ds(s, QB)],
                      eidx_hbm.at[1, pl.ds(s, QB)], QB)

    @pl.when(w == NW - 1)
    def _():
        run_batch(eidx_hbm.at[0, pl.ds(G_REAL - 20, 16)],
                  eidx_hbm.at[1, pl.ds(G_REAL - 20, 16)], 16)
        run_batch(eidx_hbm.at[0, pl.ds(G_REAL - 4, 4)],
                  eidx_hbm.at[1, pl.ds(G_REAL - 4, 4)], 4)
        for s, l in ((0, 16), (16, 16), (32, 16), (48, 12)):
            run_batch(ps_hbm.at[pl.ds(s, l)], pd_hbm.at[pl.ds(s, l)], l)

    plsc.subcore_barrier()

    @pl.when(cid == 0)
    def _():
        pltpu.sync_copy(acc_s.at[pl.ds(base, RPT)], p0_hbm.at[pl.ds(base, RPT)])
        if want_deg:
            pltpu.sync_copy(deg_s.at[pl.ds(base, RPT)],
                            g0_hbm.at[pl.ds(base, RPT)])

    @pl.when(cid == 1)
    def _():
        pltpu.sync_copy(acc_s.at[pl.ds(base, RPT)], p1_hbm.at[pl.ds(base, RPT)])
        if want_deg:
            pltpu.sync_copy(deg_s.at[pl.ds(base, RPT)],
                            g1_hbm.at[pl.ds(base, RPT)])


def _sc_out_type(want_deg):
    mat = jax.ShapeDtypeStruct((N_ACC, D), jnp.float32)
    vec = jax.ShapeDtypeStruct((N_ACC,), jnp.float32)
    out = [mat, mat]
    if want_deg:
        out += [vec, vec]
    return out


@functools.cache
def _sc_seg_sum(want_deg):
    mesh = plsc.VectorSubcoreMesh(core_axis_name="c", subcore_axis_name="s",
                                  num_cores=NC, num_subcores=NS)
    return pl.kernel(
        _sc_body_fixed(want_deg),
        out_type=_sc_out_type(want_deg),
        mesh=mesh,
        scratch_types=[
            pltpu.VMEM_SHARED((N_ACC, D), jnp.float32),
            pltpu.VMEM_SHARED((N_ACC,), jnp.float32),
            pltpu.VMEM((QB, GROUP), jnp.int32),
            pltpu.VMEM((QB, GROUP), jnp.int32),
            pltpu.VMEM((GROUP, D), jnp.float32),
            pltpu.VMEM((GROUP, D), jnp.float32),
            pltpu.VMEM((RPT,), jnp.float32),
            pltpu.VMEM((GROUP,), jnp.float32),
            pltpu.SemaphoreType.DMA,
            pltpu.SemaphoreType.DMA,
        ],
    )


_BLK = 2000


def _tc_pre_body(h_ref, ws_ref, b_ref, o_ref):
    o_ref[...] = jnp.dot(h_ref[...].astype(jnp.bfloat16),
                         ws_ref[...].astype(jnp.bfloat16),
                         preferred_element_type=jnp.float32) + b_ref[...]


def _tc_combine_body(relu, pre_ref, p0_ref, p1_ref, r_ref, wn_ref, o_ref):
    psum = p0_ref[...] + p1_ref[...]
    rcol = jnp.transpose(r_ref[...].reshape(1, _BLK))
    agg = psum * rcol
    o = pre_ref[...] + jnp.dot(agg.astype(jnp.bfloat16),
                               wn_ref[...].astype(jnp.bfloat16),
                               preferred_element_type=jnp.float32)
    if relu:
        o = jnp.maximum(o, 0.0)
    o_ref[...] = o


@functools.cache
def _tc_pre():
    mat = pl.BlockSpec((_BLK, D), lambda i: (i, 0))
    return pl.pallas_call(
        _tc_pre_body,
        grid=(N // _BLK,),
        in_specs=[mat,
                  pl.BlockSpec((D, D), lambda i: (0, 0)),
                  pl.BlockSpec((1, D), lambda i: (0, 0))],
        out_specs=mat,
        out_shape=jax.ShapeDtypeStruct((N, D), jnp.float32),
    )


@functools.cache
def _tc_combine(relu):
    mat = pl.BlockSpec((_BLK, D), lambda i: (i, 0))
    drow = pl.BlockSpec((1, 1, _BLK), lambda i: (i, 0, 0))
    wspec = pl.BlockSpec((D, D), lambda i: (0, 0))
    return pl.pallas_call(
        functools.partial(_tc_combine_body, relu),
        grid=(N // _BLK,),
        in_specs=[mat, mat, mat, drow, wspec],
        out_specs=mat,
        out_shape=jax.ShapeDtypeStruct((N, D), jnp.float32),
    )


def kernel(x, edge_index, W_self1, W_neigh1, b1, W_self2, W_neigh2, b2):
    eidx = edge_index.reshape(2, G_REAL, GROUP)
    ps = jnp.asarray(_PAD_SRC)
    pd = jnp.asarray(_PAD_DST)
    z2 = jnp.zeros((128, D), jnp.float32)
    z1 = jnp.zeros((RPT,), jnp.float32)
    o1 = jnp.ones((GROUP,), jnp.float32)

    pre1 = _tc_pre()(x, W_self1, b1.reshape(1, D))
    p0, p1, g0, g1 = _sc_seg_sum(True)(x, eidx, ps, pd, z2, z1, o1)
    rd = (1.0 / jnp.maximum(g0[:N] + g1[:N], 1.0)).reshape(N // _BLK, 1, _BLK)
    h = _tc_combine(True)(pre1, p0, p1, rd, W_neigh1)

    pre2 = _tc_pre()(h, W_self2, b2.reshape(1, D))
    q0, q1 = _sc_seg_sum(False)(h, eidx, ps, pd, z2, z1, o1)
    out = _tc_combine(False)(pre2, q0, q1, rd, W_neigh2)
    return out
